# Initial kernel scaffold; baseline (speedup 1.0000x reference)
import jax
import jax.numpy as jnp
from jax.experimental import pallas as pl


def kernel(z, dist, edge_index, en_table, radius_table, corr_table):
    raise NotImplementedError("write your pallas kernel here")


# speedup vs baseline: 215.6929x; 215.6929x over previous
import functools

import jax
import jax.numpy as jnp
from jax import lax
from jax.experimental import pallas as pl
from jax.experimental.pallas import tpu as pltpu
from jax.experimental.pallas import tpu_sc as plsc

N_NODES = 100000
N_EDGES = 3200000
NT = 104
NTP = 112

NUM_CORES = 2
NUM_SUBCORES = 16
NUM_TILES = NUM_CORES * NUM_SUBCORES
E_PER_TILE = N_EDGES // NUM_TILES
CHUNK = 800
N_CHUNKS = E_PER_TILE // CHUNK
VECS = CHUNK // 16

K0 = 7.5
K1 = 4.1
K2 = 19.09
K3 = 254.56
EPS = 1e-6

_AP = 0.3275911
_A1 = 0.254829592
_A2 = -0.284496736
_A3 = 1.421413741
_A4 = -1.453152027
_A5 = 1.061405429


def _erf(a):
    x = jnp.abs(a)
    t = 1.0 / (1.0 + _AP * x)
    poly = ((((_A5 * t + _A4) * t + _A3) * t + _A2) * t + _A1) * t
    y = 1.0 - poly * jnp.exp(-(x * x))
    return jnp.where(a >= 0.0, y, -y)


def _body(z_hbm, row_hbm, col_hbm, dist_hbm, en_hbm, rad_hbm, corr_hbm,
          out_hbm, z_v, en_v, rad_v, corr_v, dtab, rctab,
          row_v, col_v, dist_v, out_v, sem):
    wid = lax.axis_index("s") * NUM_CORES + lax.axis_index("c")

    pltpu.sync_copy(z_hbm, z_v)
    pltpu.sync_copy(en_hbm, en_v)
    pltpu.sync_copy(rad_hbm, rad_v)
    pltpu.sync_copy(corr_hbm, corr_v)

    for t in range(NTP // 16):
        s = pl.ds(t * 16, 16)
        rad_v[s] = rad_v[s] + corr_v[s]

    def build_row(zi, carry):
        idx_i = jnp.full((16,), zi, dtype=jnp.int32)
        en_i = plsc.load_gather(en_v, [idx_i])
        r_i = plsc.load_gather(rad_v, [idx_i])
        for t in range(NTP // 16):
            zj = t * 16 + jax.lax.iota(jnp.int32, 16)
            en_j = plsc.load_gather(en_v, [zj])
            r_j = plsc.load_gather(rad_v, [zj])
            d = jnp.abs(en_i - en_j) + K2
            delta = (0.5 * K1) * jnp.exp(d * d * (-1.0 / K3))
            base = zi * NTP + t * 16
            dtab[pl.ds(base, 16)] = delta
            rctab[pl.ds(base, 16)] = r_i + r_j
        return carry

    lax.fori_loop(0, NT, build_row, 0)

    tile_base = wid * E_PER_TILE

    def chunk_body(ci, carry):
        base = tile_base + ci * CHUNK
        pltpu.sync_copy(row_hbm.at[pl.ds(base, CHUNK)], row_v)
        pltpu.sync_copy(col_hbm.at[pl.ds(base, CHUNK)], col_v)
        pltpu.sync_copy(dist_hbm.at[pl.ds(base, CHUNK)], dist_v)

        def vec_body(i, c2):
            s = pl.ds(i * 16, 16)
            r = row_v[s]
            c = col_v[s]
            d = dist_v[s]
            z_i = plsc.load_gather(z_v, [r])
            z_j = plsc.load_gather(z_v, [c])
            p = z_i * NTP + z_j
            delta = plsc.load_gather(dtab, [p])
            rc = plsc.load_gather(rctab, [p])
            a = (-K0) * (d - rc) / (rc + EPS)
            out_v[s] = delta * (1.0 + _erf(a))
            return c2

        lax.fori_loop(0, VECS, vec_body, 0)
        pltpu.sync_copy(out_v, out_hbm.at[pl.ds(base, CHUNK)])
        return carry

    lax.fori_loop(0, N_CHUNKS, chunk_body, 0)


_mesh = plsc.VectorSubcoreMesh(core_axis_name="c", subcore_axis_name="s")

_edge_kernel = functools.partial(
    pl.kernel,
    out_type=jax.ShapeDtypeStruct((N_EDGES,), jnp.float32),
    mesh=_mesh,
    compiler_params=pltpu.CompilerParams(needs_layout_passes=False),
    scratch_types=[
        pltpu.VMEM((N_NODES,), jnp.int32),
        pltpu.VMEM((NTP,), jnp.float32),
        pltpu.VMEM((NTP,), jnp.float32),
        pltpu.VMEM((NTP,), jnp.float32),
        pltpu.VMEM((NT * NTP,), jnp.float32),
        pltpu.VMEM((NT * NTP,), jnp.float32),
        pltpu.VMEM((CHUNK,), jnp.int32),
        pltpu.VMEM((CHUNK,), jnp.int32),
        pltpu.VMEM((CHUNK,), jnp.float32),
        pltpu.VMEM((CHUNK,), jnp.float32),
        pltpu.SemaphoreType.DMA,
    ],
)(_body)


def kernel(z, dist, edge_index, en_table, radius_table, corr_table):
    row = edge_index[0]
    col = edge_index[1]
    en = jnp.pad(en_table[:, 0], (0, NTP - NT), constant_values=1.0)
    rad = jnp.pad(radius_table[:, 0], (0, NTP - NT), constant_values=1.0)
    corr = jnp.pad(corr_table[:, 0], (0, NTP - NT), constant_values=0.0)
    out = _edge_kernel(z, row, col, dist, en, rad, corr)
    return out[:, None]


# speedup vs baseline: 253.6318x; 1.1759x over previous
import functools

import jax
import jax.numpy as jnp
from jax import lax
from jax.experimental import pallas as pl
from jax.experimental.pallas import tpu as pltpu
from jax.experimental.pallas import tpu_sc as plsc

N_NODES = 100000
N_EDGES = 3200000
NT = 104
NTP = 112

NUM_CORES = 2
NUM_SUBCORES = 16
NUM_TILES = NUM_CORES * NUM_SUBCORES
E_PER_TILE = N_EDGES // NUM_TILES
CHUNK = 400
N_CHUNKS = E_PER_TILE // CHUNK
N_PAIRS = N_CHUNKS // 2
VECS = CHUNK // 16

K0 = 7.5
K1 = 4.1
K2 = 19.09
K3 = 254.56
EPS = 1e-6

_AP = 0.3275911
_A1 = 0.254829592
_A2 = -0.284496736
_A3 = 1.421413741
_A4 = -1.453152027
_A5 = 1.061405429


def _erf(a):
    x = jnp.abs(a)
    t = 1.0 / (1.0 + _AP * x)
    poly = ((((_A5 * t + _A4) * t + _A3) * t + _A2) * t + _A1) * t
    y = 1.0 - poly * jnp.exp(-(x * x))
    return jnp.where(a >= 0.0, y, -y)


def _body(z_hbm, row_hbm, col_hbm, dist_hbm, en_hbm, rad_hbm, corr_hbm,
          out_hbm, z_v, en_v, rad_v, corr_v, dtab, rctab,
          row0, row1, col0, col1, dist0, dist1, out0, out1,
          sem_in0, sem_in1, sem_out0, sem_out1):
    wid = lax.axis_index("s") * NUM_CORES + lax.axis_index("c")
    rows = (row0, row1)
    cols = (col0, col1)
    dists = (dist0, dist1)
    outs = (out0, out1)
    sems_in = (sem_in0, sem_in1)
    sems_out = (sem_out0, sem_out1)

    pltpu.sync_copy(z_hbm, z_v)
    pltpu.sync_copy(en_hbm, en_v)
    pltpu.sync_copy(rad_hbm, rad_v)
    pltpu.sync_copy(corr_hbm, corr_v)

    for t in range(NTP // 16):
        s = pl.ds(t * 16, 16)
        rad_v[s] = rad_v[s] + corr_v[s]

    def build_row(zi, carry):
        idx_i = jnp.full((16,), zi, dtype=jnp.int32)
        en_i = plsc.load_gather(en_v, [idx_i])
        r_i = plsc.load_gather(rad_v, [idx_i])
        for t in range(NTP // 16):
            zj = t * 16 + jax.lax.iota(jnp.int32, 16)
            en_j = plsc.load_gather(en_v, [zj])
            r_j = plsc.load_gather(rad_v, [zj])
            d = jnp.abs(en_i - en_j) + K2
            delta = (0.5 * K1) * jnp.exp(d * d * (-1.0 / K3))
            base = zi * NTP + t * 16
            dtab[pl.ds(base, 16)] = delta
            rctab[pl.ds(base, 16)] = r_i + r_j
        return carry

    lax.fori_loop(0, NT, build_row, 0)

    tile_base = wid * E_PER_TILE

    def start_in(ci, b):
        base = tile_base + ci * CHUNK
        pltpu.async_copy(row_hbm.at[pl.ds(base, CHUNK)], rows[b], sems_in[b])
        pltpu.async_copy(col_hbm.at[pl.ds(base, CHUNK)], cols[b], sems_in[b])
        pltpu.async_copy(dist_hbm.at[pl.ds(base, CHUNK)], dists[b], sems_in[b])

    def wait_in(b):
        pltpu.make_async_copy(row_hbm.at[pl.ds(0, CHUNK)], rows[b], sems_in[b]).wait()
        pltpu.make_async_copy(col_hbm.at[pl.ds(0, CHUNK)], cols[b], sems_in[b]).wait()
        pltpu.make_async_copy(dist_hbm.at[pl.ds(0, CHUNK)], dists[b], sems_in[b]).wait()

    def start_out(ci, b):
        base = tile_base + ci * CHUNK
        pltpu.async_copy(outs[b], out_hbm.at[pl.ds(base, CHUNK)], sems_out[b])

    def wait_out(b):
        pltpu.make_async_copy(outs[b], out_hbm.at[pl.ds(0, CHUNK)], sems_out[b]).wait()

    def compute(b):
        row_b, col_b, dist_b, out_b = rows[b], cols[b], dists[b], outs[b]
        for i in range(VECS):
            s = pl.ds(i * 16, 16)
            r = row_b[s]
            c = col_b[s]
            d = dist_b[s]
            z_i = plsc.load_gather(z_v, [r])
            z_j = plsc.load_gather(z_v, [c])
            p = z_i * NTP + z_j
            delta = plsc.load_gather(dtab, [p])
            rc = plsc.load_gather(rctab, [p])
            a = (-K0) * (d - rc) / (rc + EPS)
            out_b[s] = delta * (1.0 + _erf(a))

    start_in(0, 0)

    def pair_body(g, carry):
        c0 = 2 * g
        start_in(c0 + 1, 1)
        wait_in(0)

        @pl.when(g > 0)
        def _():
            wait_out(0)

        compute(0)
        start_out(c0, 0)

        @pl.when(g + 1 < N_PAIRS)
        def _():
            start_in(c0 + 2, 0)

        wait_in(1)

        @pl.when(g > 0)
        def _():
            wait_out(1)

        compute(1)
        start_out(c0 + 1, 1)
        return carry

    lax.fori_loop(0, N_PAIRS, pair_body, 0)
    wait_out(0)
    wait_out(1)


_mesh = plsc.VectorSubcoreMesh(core_axis_name="c", subcore_axis_name="s")

_edge_kernel = functools.partial(
    pl.kernel,
    out_type=jax.ShapeDtypeStruct((N_EDGES,), jnp.float32),
    mesh=_mesh,
    compiler_params=pltpu.CompilerParams(needs_layout_passes=False),
    scratch_types=[
        pltpu.VMEM((N_NODES,), jnp.int32),
        pltpu.VMEM((NTP,), jnp.float32),
        pltpu.VMEM((NTP,), jnp.float32),
        pltpu.VMEM((NTP,), jnp.float32),
        pltpu.VMEM((NT * NTP,), jnp.float32),
        pltpu.VMEM((NT * NTP,), jnp.float32),
        pltpu.VMEM((CHUNK,), jnp.int32),
        pltpu.VMEM((CHUNK,), jnp.int32),
        pltpu.VMEM((CHUNK,), jnp.int32),
        pltpu.VMEM((CHUNK,), jnp.int32),
        pltpu.VMEM((CHUNK,), jnp.float32),
        pltpu.VMEM((CHUNK,), jnp.float32),
        pltpu.VMEM((CHUNK,), jnp.float32),
        pltpu.VMEM((CHUNK,), jnp.float32),
        pltpu.SemaphoreType.DMA,
        pltpu.SemaphoreType.DMA,
        pltpu.SemaphoreType.DMA,
        pltpu.SemaphoreType.DMA,
    ],
)(_body)


def kernel(z, dist, edge_index, en_table, radius_table, corr_table):
    row = edge_index[0]
    col = edge_index[1]
    en = jnp.pad(en_table[:, 0], (0, NTP - NT), constant_values=1.0)
    rad = jnp.pad(radius_table[:, 0], (0, NTP - NT), constant_values=1.0)
    corr = jnp.pad(corr_table[:, 0], (0, NTP - NT), constant_values=0.0)
    out = _edge_kernel(z, row, col, dist, en, rad, corr)
    return out[:, None]


# speedup vs baseline: 689.0827x; 2.7169x over previous
import functools

import jax
import jax.numpy as jnp
from jax import lax
from jax.experimental import pallas as pl
from jax.experimental.pallas import tpu as pltpu
from jax.experimental.pallas import tpu_sc as plsc

N_NODES = 100000
N_EDGES = 3200000
NT = 104
NTP = 112

NUM_CORES = 2
NUM_SUBCORES = 16
NUM_TILES = NUM_CORES * NUM_SUBCORES
E_PER_TILE = N_EDGES // NUM_TILES
CHUNK = 400
N_CHUNKS = E_PER_TILE // CHUNK
N_PAIRS = N_CHUNKS // 2
VECS = CHUNK // 16

K0 = 7.5
K1 = 4.1
K2 = 19.09
K3 = 254.56
EPS = 1e-6

_AP = 0.3275911
_A1 = 0.254829592
_A2 = -0.284496736
_A3 = 1.421413741
_A4 = -1.453152027
_A5 = 1.061405429


def _erf(a):
    x = jnp.abs(a)
    t = 1.0 / (1.0 + _AP * x)
    poly = ((((_A5 * t + _A4) * t + _A3) * t + _A2) * t + _A1) * t
    y = 1.0 - poly * jnp.exp(-(x * x))
    return jnp.where(a >= 0.0, y, -y)


def _body(z_hbm, row_hbm, col_hbm, dist_hbm, en_hbm, rad_hbm, corr_hbm,
          out_hbm, z_v, en_v, rad_v, corr_v, dtab, rctab,
          row0, row1, col0, col1, dist0, dist1, out0, out1,
          sem_in0, sem_in1, sem_out0, sem_out1):
    wid = lax.axis_index("s") * NUM_CORES + lax.axis_index("c")
    rows = (row0, row1)
    cols = (col0, col1)
    dists = (dist0, dist1)
    outs = (out0, out1)
    sems_in = (sem_in0, sem_in1)
    sems_out = (sem_out0, sem_out1)

    pltpu.sync_copy(z_hbm, z_v)
    pltpu.sync_copy(en_hbm, en_v)
    pltpu.sync_copy(rad_hbm, rad_v)
    pltpu.sync_copy(corr_hbm, corr_v)

    for t in range(NTP // 16):
        s = pl.ds(t * 16, 16)
        rad_v[s] = rad_v[s] + corr_v[s]

    def build_row(zi, carry):
        idx_i = jnp.full((16,), zi, dtype=jnp.int32)
        en_i = plsc.load_gather(en_v, [idx_i])
        r_i = plsc.load_gather(rad_v, [idx_i])
        for t in range(NTP // 16):
            zj = t * 16 + jax.lax.iota(jnp.int32, 16)
            en_j = plsc.load_gather(en_v, [zj])
            r_j = plsc.load_gather(rad_v, [zj])
            d = jnp.abs(en_i - en_j) + K2
            delta = (0.5 * K1) * jnp.exp(d * d * (-1.0 / K3))
            base = zi * NTP + t * 16
            dtab[pl.ds(base, 16)] = delta
            rctab[pl.ds(base, 16)] = r_i + r_j
        return carry

    lax.fori_loop(0, NT, build_row, 0)

    tile_base = wid * E_PER_TILE

    def start_in(ci, b):
        base = tile_base + ci * CHUNK
        pltpu.async_copy(row_hbm.at[pl.ds(base, CHUNK)], rows[b], sems_in[b])
        pltpu.async_copy(col_hbm.at[pl.ds(base, CHUNK)], cols[b], sems_in[b])
        pltpu.async_copy(dist_hbm.at[pl.ds(base, CHUNK)], dists[b], sems_in[b])

    def wait_in(b):
        pltpu.make_async_copy(row_hbm.at[pl.ds(0, CHUNK)], rows[b], sems_in[b]).wait()
        pltpu.make_async_copy(col_hbm.at[pl.ds(0, CHUNK)], cols[b], sems_in[b]).wait()
        pltpu.make_async_copy(dist_hbm.at[pl.ds(0, CHUNK)], dists[b], sems_in[b]).wait()

    def start_out(ci, b):
        base = tile_base + ci * CHUNK
        pltpu.async_copy(outs[b], out_hbm.at[pl.ds(base, CHUNK)], sems_out[b])

    def wait_out(b):
        pltpu.make_async_copy(outs[b], out_hbm.at[pl.ds(0, CHUNK)], sems_out[b]).wait()

    def compute(b):
        row_b, col_b, dist_b, out_b = rows[b], cols[b], dists[b], outs[b]

        @plsc.parallel_loop(0, CHUNK, 16, unroll=5)
        def _(i):
            s = pl.ds(i, 16)
            r = row_b[s]
            c = col_b[s]
            d = dist_b[s]
            z_i = plsc.load_gather(z_v, [r])
            z_j = plsc.load_gather(z_v, [c])
            p = z_i * NTP + z_j
            delta = plsc.load_gather(dtab, [p])
            rc = plsc.load_gather(rctab, [p])
            a = (-K0) * (d - rc) / (rc + EPS)
            out_b[s] = delta * (1.0 + _erf(a))

    start_in(0, 0)

    def pair_body(g, carry):
        c0 = 2 * g
        start_in(c0 + 1, 1)
        wait_in(0)

        @pl.when(g > 0)
        def _():
            wait_out(0)

        compute(0)
        start_out(c0, 0)

        @pl.when(g + 1 < N_PAIRS)
        def _():
            start_in(c0 + 2, 0)

        wait_in(1)

        @pl.when(g > 0)
        def _():
            wait_out(1)

        compute(1)
        start_out(c0 + 1, 1)
        return carry

    lax.fori_loop(0, N_PAIRS, pair_body, 0)
    wait_out(0)
    wait_out(1)


_mesh = plsc.VectorSubcoreMesh(core_axis_name="c", subcore_axis_name="s")

_edge_kernel = functools.partial(
    pl.kernel,
    out_type=jax.ShapeDtypeStruct((N_EDGES,), jnp.float32),
    mesh=_mesh,
    compiler_params=pltpu.CompilerParams(needs_layout_passes=False),
    scratch_types=[
        pltpu.VMEM((N_NODES,), jnp.int32),
        pltpu.VMEM((NTP,), jnp.float32),
        pltpu.VMEM((NTP,), jnp.float32),
        pltpu.VMEM((NTP,), jnp.float32),
        pltpu.VMEM((NT * NTP,), jnp.float32),
        pltpu.VMEM((NT * NTP,), jnp.float32),
        pltpu.VMEM((CHUNK,), jnp.int32),
        pltpu.VMEM((CHUNK,), jnp.int32),
        pltpu.VMEM((CHUNK,), jnp.int32),
        pltpu.VMEM((CHUNK,), jnp.int32),
        pltpu.VMEM((CHUNK,), jnp.float32),
        pltpu.VMEM((CHUNK,), jnp.float32),
        pltpu.VMEM((CHUNK,), jnp.float32),
        pltpu.VMEM((CHUNK,), jnp.float32),
        pltpu.SemaphoreType.DMA,
        pltpu.SemaphoreType.DMA,
        pltpu.SemaphoreType.DMA,
        pltpu.SemaphoreType.DMA,
    ],
)(_body)


def kernel(z, dist, edge_index, en_table, radius_table, corr_table):
    row = edge_index[0]
    col = edge_index[1]
    en = jnp.pad(en_table[:, 0], (0, NTP - NT), constant_values=1.0)
    rad = jnp.pad(radius_table[:, 0], (0, NTP - NT), constant_values=1.0)
    corr = jnp.pad(corr_table[:, 0], (0, NTP - NT), constant_values=0.0)
    out = _edge_kernel(z, row, col, dist, en, rad, corr)
    return out[:, None]


# speedup vs baseline: 769.7120x; 1.1170x over previous
import functools

import jax
import jax.numpy as jnp
import numpy as np
from jax import lax
from jax.experimental import pallas as pl
from jax.experimental.pallas import tpu as pltpu
from jax.experimental.pallas import tpu_sc as plsc

N_NODES = 100000
N_EDGES = 3200000
NT = 104
NTP = 112

NUM_CORES = 2
NUM_SUBCORES = 16
NUM_TILES = NUM_CORES * NUM_SUBCORES
E_PER_TILE = N_EDGES // NUM_TILES
CHUNK = 400
N_CHUNKS = E_PER_TILE // CHUNK
N_PAIRS = N_CHUNKS // 2

K0 = 7.5
K1 = 4.1
K2 = 19.09
K3 = 254.56
EPS = 1e-6

_AP = 0.47047
_A1 = 0.3480242
_A2 = -0.0958798
_A3 = 0.7478556

_SIGN = np.int32(-2147483648)


def _body(z_hbm, row_hbm, col_hbm, dist_hbm, en_hbm, rad_hbm, corr_hbm,
          out_hbm, z_v, en_v, rad_v, corr_v, dtab, atab,
          row0, row1, col0, col1, dist0, dist1, out0, out1,
          sem_z, sem_in0, sem_in1, sem_out0, sem_out1):
    wid = lax.axis_index("s") * NUM_CORES + lax.axis_index("c")
    rows = (row0, row1)
    cols = (col0, col1)
    dists = (dist0, dist1)
    outs = (out0, out1)
    sems_in = (sem_in0, sem_in1)
    sems_out = (sem_out0, sem_out1)

    z_copy = pltpu.async_copy(z_hbm, z_v, sem_z)
    pltpu.sync_copy(en_hbm, en_v)
    pltpu.sync_copy(rad_hbm, rad_v)
    pltpu.sync_copy(corr_hbm, corr_v)

    for t in range(NTP // 16):
        s = pl.ds(t * 16, 16)
        rad_v[s] = rad_v[s] + corr_v[s]

    @plsc.parallel_loop(0, NT, 1, unroll=2)
    def _build(zi):
        idx_i = jnp.full((16,), zi, dtype=jnp.int32)
        en_i = plsc.load_gather(en_v, [idx_i])
        r_i = plsc.load_gather(rad_v, [idx_i])
        for t in range(NTP // 16):
            zj = t * 16 + jax.lax.iota(jnp.int32, 16)
            en_j = plsc.load_gather(en_v, [zj])
            r_j = plsc.load_gather(rad_v, [zj])
            d = jnp.abs(en_i - en_j) + K2
            delta = (0.5 * K1) * jnp.exp(d * d * (-1.0 / K3))
            base = zi * NTP + t * 16
            dtab[pl.ds(base, 16)] = delta
            atab[pl.ds(base, 16)] = (-K0) / (r_i + r_j + EPS)

    tile_base = wid * E_PER_TILE

    def start_in(ci, b):
        base = tile_base + ci * CHUNK
        pltpu.async_copy(row_hbm.at[pl.ds(base, CHUNK)], rows[b], sems_in[b])
        pltpu.async_copy(col_hbm.at[pl.ds(base, CHUNK)], cols[b], sems_in[b])
        pltpu.async_copy(dist_hbm.at[pl.ds(base, CHUNK)], dists[b], sems_in[b])

    def wait_in(b):
        pltpu.make_async_copy(row_hbm.at[pl.ds(0, CHUNK)], rows[b], sems_in[b]).wait()
        pltpu.make_async_copy(col_hbm.at[pl.ds(0, CHUNK)], cols[b], sems_in[b]).wait()
        pltpu.make_async_copy(dist_hbm.at[pl.ds(0, CHUNK)], dists[b], sems_in[b]).wait()

    def start_out(ci, b):
        base = tile_base + ci * CHUNK
        pltpu.async_copy(outs[b], out_hbm.at[pl.ds(base, CHUNK)], sems_out[b])

    def wait_out(b):
        pltpu.make_async_copy(outs[b], out_hbm.at[pl.ds(0, CHUNK)], sems_out[b]).wait()

    def compute(b):
        row_b, col_b, dist_b, out_b = rows[b], cols[b], dists[b], outs[b]

        @plsc.parallel_loop(0, CHUNK, 16, unroll=5)
        def _(i):
            s = pl.ds(i, 16)
            r = row_b[s]
            c = col_b[s]
            d = dist_b[s]
            z_i = plsc.load_gather(z_v, [r])
            z_j = plsc.load_gather(z_v, [c])
            p = z_i * NTP + z_j
            delta = plsc.load_gather(dtab, [p])
            aa = plsc.load_gather(atab, [p])
            a = aa * d + K0
            x = jnp.abs(a)
            t = 1.0 / (1.0 + _AP * x)
            poly = ((_A3 * t + _A2) * t + _A1) * t
            y = 1.0 - poly * jnp.exp(-(a * a))
            sbit = lax.bitcast_convert_type(a, jnp.int32) & _SIGN
            erf = lax.bitcast_convert_type(
                lax.bitcast_convert_type(y, jnp.int32) ^ sbit, jnp.float32)
            out_b[s] = delta * (1.0 + erf)

    start_in(0, 0)
    z_copy.wait()

    def pair_body(g, carry):
        c0 = 2 * g
        start_in(c0 + 1, 1)
        wait_in(0)

        @pl.when(g > 0)
        def _():
            wait_out(0)

        compute(0)
        start_out(c0, 0)

        @pl.when(g + 1 < N_PAIRS)
        def _():
            start_in(c0 + 2, 0)

        wait_in(1)

        @pl.when(g > 0)
        def _():
            wait_out(1)

        compute(1)
        start_out(c0 + 1, 1)
        return carry

    lax.fori_loop(0, N_PAIRS, pair_body, 0)
    wait_out(0)
    wait_out(1)


_mesh = plsc.VectorSubcoreMesh(core_axis_name="c", subcore_axis_name="s")

_edge_kernel = functools.partial(
    pl.kernel,
    out_type=jax.ShapeDtypeStruct((N_EDGES,), jnp.float32),
    mesh=_mesh,
    compiler_params=pltpu.CompilerParams(needs_layout_passes=False),
    scratch_types=[
        pltpu.VMEM((N_NODES,), jnp.int32),
        pltpu.VMEM((NTP,), jnp.float32),
        pltpu.VMEM((NTP,), jnp.float32),
        pltpu.VMEM((NTP,), jnp.float32),
        pltpu.VMEM((NT * NTP,), jnp.float32),
        pltpu.VMEM((NT * NTP,), jnp.float32),
        pltpu.VMEM((CHUNK,), jnp.int32),
        pltpu.VMEM((CHUNK,), jnp.int32),
        pltpu.VMEM((CHUNK,), jnp.int32),
        pltpu.VMEM((CHUNK,), jnp.int32),
        pltpu.VMEM((CHUNK,), jnp.float32),
        pltpu.VMEM((CHUNK,), jnp.float32),
        pltpu.VMEM((CHUNK,), jnp.float32),
        pltpu.VMEM((CHUNK,), jnp.float32),
        pltpu.SemaphoreType.DMA,
        pltpu.SemaphoreType.DMA,
        pltpu.SemaphoreType.DMA,
        pltpu.SemaphoreType.DMA,
        pltpu.SemaphoreType.DMA,
    ],
)(_body)


def kernel(z, dist, edge_index, en_table, radius_table, corr_table):
    row = edge_index[0]
    col = edge_index[1]
    en = jnp.pad(en_table[:, 0], (0, NTP - NT), constant_values=1.0)
    rad = jnp.pad(radius_table[:, 0], (0, NTP - NT), constant_values=1.0)
    corr = jnp.pad(corr_table[:, 0], (0, NTP - NT), constant_values=0.0)
    out = _edge_kernel(z, row, col, dist, en, rad, corr)
    return out[:, None]


# speedup vs baseline: 804.5117x; 1.0452x over previous
import functools

import jax
import jax.numpy as jnp
import numpy as np
from jax import lax
from jax.experimental import pallas as pl
from jax.experimental.pallas import tpu as pltpu
from jax.experimental.pallas import tpu_sc as plsc

N_NODES = 100000
N_EDGES = 3200000
NT = 104
NTP = 112

NUM_CORES = 2
NUM_SUBCORES = 16
NUM_TILES = NUM_CORES * NUM_SUBCORES
E_PER_TILE = N_EDGES // NUM_TILES
CHUNK = 800
N_CHUNKS = E_PER_TILE // CHUNK
N_PAIRS = N_CHUNKS // 2

K0 = 7.5
K1 = 4.1
K2 = 19.09
K3 = 254.56
EPS = 1e-6

_AP = 0.47047
_A1 = 0.3480242
_A2 = -0.0958798
_A3 = 0.7478556

_SIGN = np.int32(-2147483648)
_HI16 = np.int32(-65536)
_LO16 = np.int32(65535)
_RND = np.int32(32768)
_ASC = 65535.0 / 64.0


def _body(z_hbm, row_hbm, col_hbm, dist_hbm, en_hbm, rad_hbm, corr_hbm,
          out_hbm, z_v, en_v, rad_v, corr_v, ptab,
          row0, row1, col0, col1, dist0, dist1, out0, out1,
          sem_z, sem_r0, sem_r1, sem_c0, sem_c1, sem_d0, sem_d1,
          sem_out0, sem_out1):
    wid = lax.axis_index("s") * NUM_CORES + lax.axis_index("c")
    rows = (row0, row1)
    cols = (col0, col1)
    dists = (dist0, dist1)
    outs = (out0, out1)
    sems_r = (sem_r0, sem_r1)
    sems_c = (sem_c0, sem_c1)
    sems_d = (sem_d0, sem_d1)
    sems_out = (sem_out0, sem_out1)

    z_copy = pltpu.async_copy(z_hbm, z_v, sem_z)
    pltpu.sync_copy(en_hbm, en_v)
    pltpu.sync_copy(rad_hbm, rad_v)
    pltpu.sync_copy(corr_hbm, corr_v)

    for t in range(NTP // 16):
        s = pl.ds(t * 16, 16)
        rad_v[s] = rad_v[s] + corr_v[s]

    @plsc.parallel_loop(0, NT, 1, unroll=2)
    def _build(zi):
        idx_i = jnp.full((16,), zi, dtype=jnp.int32)
        en_i = plsc.load_gather(en_v, [idx_i])
        r_i = plsc.load_gather(rad_v, [idx_i])
        for t in range(NTP // 16):
            zj = t * 16 + jax.lax.iota(jnp.int32, 16)
            en_j = plsc.load_gather(en_v, [zj])
            r_j = plsc.load_gather(rad_v, [zj])
            d = jnp.abs(en_i - en_j) + K2
            delta = (0.5 * K1) * jnp.exp(d * d * (-1.0 / K3))
            db = (lax.bitcast_convert_type(delta, jnp.int32) + _RND) & _HI16
            av = (-K0) / (r_i + r_j + EPS)
            enc = jnp.clip((av + 64.0) * _ASC + 0.5, 0.0, 65535.0).astype(jnp.int32)
            base = zi * NTP + t * 16
            ptab[pl.ds(base, 16)] = db | enc

    tile_base = wid * E_PER_TILE

    def start_in(ci, b):
        base = tile_base + ci * CHUNK
        pltpu.async_copy(row_hbm.at[pl.ds(base, CHUNK)], rows[b], sems_r[b])
        pltpu.async_copy(col_hbm.at[pl.ds(base, CHUNK)], cols[b], sems_c[b])
        pltpu.async_copy(dist_hbm.at[pl.ds(base, CHUNK)], dists[b], sems_d[b])

    def wait_in(b):
        pltpu.make_async_copy(row_hbm.at[pl.ds(0, CHUNK)], rows[b], sems_r[b]).wait()
        pltpu.make_async_copy(col_hbm.at[pl.ds(0, CHUNK)], cols[b], sems_c[b]).wait()
        pltpu.make_async_copy(dist_hbm.at[pl.ds(0, CHUNK)], dists[b], sems_d[b]).wait()

    def start_out(ci, b):
        base = tile_base + ci * CHUNK
        pltpu.async_copy(outs[b], out_hbm.at[pl.ds(base, CHUNK)], sems_out[b])

    def wait_out(b):
        pltpu.make_async_copy(outs[b], out_hbm.at[pl.ds(0, CHUNK)], sems_out[b]).wait()

    def compute(b):
        row_b, col_b, dist_b, out_b = rows[b], cols[b], dists[b], outs[b]

        @plsc.parallel_loop(0, CHUNK, 16, unroll=5)
        def _(i):
            s = pl.ds(i, 16)
            r = row_b[s]
            c = col_b[s]
            d = dist_b[s]
            wi = plsc.load_gather(z_v, [lax.shift_right_logical(r, 1)])
            wj = plsc.load_gather(z_v, [lax.shift_right_logical(c, 1)])
            z_i = lax.shift_right_logical(wi, lax.shift_left(r & 1, 4)) & _LO16
            z_j = lax.shift_right_logical(wj, lax.shift_left(c & 1, 4)) & _LO16
            p = z_i * NTP + z_j
            w = plsc.load_gather(ptab, [p])
            delta = lax.bitcast_convert_type(w & _HI16, jnp.float32)
            aa = (w & _LO16).astype(jnp.float32) * (1.0 / _ASC) - 64.0
            a = aa * d + K0
            x = jnp.abs(a)
            t = 1.0 / (1.0 + _AP * x)
            poly = ((_A3 * t + _A2) * t + _A1) * t
            y = 1.0 - poly * jnp.exp(-(a * a))
            sbit = lax.bitcast_convert_type(a, jnp.int32) & _SIGN
            erf = lax.bitcast_convert_type(
                lax.bitcast_convert_type(y, jnp.int32) ^ sbit, jnp.float32)
            out_b[s] = delta * (1.0 + erf)

    start_in(0, 0)
    z_copy.wait()

    def pair_body(g, carry):
        c0 = 2 * g
        start_in(c0 + 1, 1)
        wait_in(0)

        @pl.when(g > 0)
        def _():
            wait_out(0)

        compute(0)
        start_out(c0, 0)
        start_in(c0 + 2, 0)
        wait_in(1)

        @pl.when(g > 0)
        def _():
            wait_out(1)

        compute(1)
        start_out(c0 + 1, 1)
        return carry

    lax.fori_loop(0, N_PAIRS, pair_body, 0)

    wait_in(0)
    wait_out(0)
    compute(0)
    start_out(N_CHUNKS - 1, 0)
    wait_out(0)
    wait_out(1)


_mesh = plsc.VectorSubcoreMesh(core_axis_name="c", subcore_axis_name="s")

_edge_kernel = functools.partial(
    pl.kernel,
    out_type=jax.ShapeDtypeStruct((N_EDGES,), jnp.float32),
    mesh=_mesh,
    compiler_params=pltpu.CompilerParams(needs_layout_passes=False),
    scratch_types=[
        pltpu.VMEM((N_NODES // 2,), jnp.int32),
        pltpu.VMEM((NTP,), jnp.float32),
        pltpu.VMEM((NTP,), jnp.float32),
        pltpu.VMEM((NTP,), jnp.float32),
        pltpu.VMEM((NT * NTP,), jnp.int32),
        pltpu.VMEM((CHUNK,), jnp.int32),
        pltpu.VMEM((CHUNK,), jnp.int32),
        pltpu.VMEM((CHUNK,), jnp.int32),
        pltpu.VMEM((CHUNK,), jnp.int32),
        pltpu.VMEM((CHUNK,), jnp.float32),
        pltpu.VMEM((CHUNK,), jnp.float32),
        pltpu.VMEM((CHUNK,), jnp.float32),
        pltpu.VMEM((CHUNK,), jnp.float32),
        pltpu.SemaphoreType.DMA,
        pltpu.SemaphoreType.DMA,
        pltpu.SemaphoreType.DMA,
        pltpu.SemaphoreType.DMA,
        pltpu.SemaphoreType.DMA,
        pltpu.SemaphoreType.DMA,
        pltpu.SemaphoreType.DMA,
        pltpu.SemaphoreType.DMA,
        pltpu.SemaphoreType.DMA,
    ],
)(_body)


def kernel(z, dist, edge_index, en_table, radius_table, corr_table):
    row = edge_index[0]
    col = edge_index[1]
    zp = z[0::2] | (z[1::2] << 16)
    en = jnp.pad(en_table[:, 0], (0, NTP - NT), constant_values=1.0)
    rad = jnp.pad(radius_table[:, 0], (0, NTP - NT), constant_values=1.0)
    corr = jnp.pad(corr_table[:, 0], (0, NTP - NT), constant_values=0.0)
    out = _edge_kernel(zp, row, col, dist, en, rad, corr)
    return out[:, None]


# speedup vs baseline: 972.5007x; 1.2088x over previous
import functools

import jax
import jax.numpy as jnp
import numpy as np
from jax import lax
from jax.experimental import pallas as pl
from jax.experimental.pallas import tpu as pltpu
from jax.experimental.pallas import tpu_sc as plsc

N_NODES = 100000
N_EDGES = 3200000
NT = 104
NTP = 112

NUM_CORES = 2
NUM_SUBCORES = 16
NUM_TILES = NUM_CORES * NUM_SUBCORES
E_PER_TILE = N_EDGES // NUM_TILES
CHUNK = 800
N_CHUNKS = E_PER_TILE // CHUNK
N_PAIRS = N_CHUNKS // 2

K0 = 7.5
K1 = 4.1
K2 = 19.09
K3 = 254.56
EPS = 1e-6

_AP = 0.47047
_A1 = 0.3480242
_A2 = -0.0958798
_A3 = 0.7478556

_SIGN = np.int32(-2147483648)


def _body(z_hbm, row_hbm, col_hbm, dist_hbm, en_hbm, rad_hbm, corr_hbm,
          out_hbm, z_v, en_v, rad_v, corr_v, dtab, atab,
          row0, row1, col0, col1, dist0, dist1, out0, out1,
          sem_z, sem_r0, sem_r1, sem_c0, sem_c1, sem_d0, sem_d1,
          sem_out0, sem_out1):
    wid = lax.axis_index("s") * NUM_CORES + lax.axis_index("c")
    rows = (row0, row1)
    cols = (col0, col1)
    dists = (dist0, dist1)
    outs = (out0, out1)
    sems_r = (sem_r0, sem_r1)
    sems_c = (sem_c0, sem_c1)
    sems_d = (sem_d0, sem_d1)
    sems_out = (sem_out0, sem_out1)

    z_copy = pltpu.async_copy(z_hbm, z_v, sem_z)
    pltpu.sync_copy(en_hbm, en_v)
    pltpu.sync_copy(rad_hbm, rad_v)
    pltpu.sync_copy(corr_hbm, corr_v)

    for t in range(NTP // 16):
        s = pl.ds(t * 16, 16)
        rad_v[s] = rad_v[s] + corr_v[s]

    @plsc.parallel_loop(0, NT, 1, unroll=2)
    def _build(zi):
        idx_i = jnp.full((16,), zi, dtype=jnp.int32)
        en_i = plsc.load_gather(en_v, [idx_i])
        r_i = plsc.load_gather(rad_v, [idx_i])
        for t in range(NTP // 16):
            zj = t * 16 + jax.lax.iota(jnp.int32, 16)
            en_j = plsc.load_gather(en_v, [zj])
            r_j = plsc.load_gather(rad_v, [zj])
            d = jnp.abs(en_i - en_j) + K2
            delta = (0.5 * K1) * jnp.exp(d * d * (-1.0 / K3))
            base = zi * NTP + t * 16
            dtab[pl.ds(base, 16)] = delta
            atab[pl.ds(base, 16)] = (-K0) / (r_i + r_j + EPS)

    tile_base = wid * E_PER_TILE

    def start_in(ci, b):
        base = tile_base + ci * CHUNK
        pltpu.async_copy(row_hbm.at[pl.ds(base, CHUNK)], rows[b], sems_r[b])
        pltpu.async_copy(col_hbm.at[pl.ds(base, CHUNK)], cols[b], sems_c[b])
        pltpu.async_copy(dist_hbm.at[pl.ds(base, CHUNK)], dists[b], sems_d[b])

    def wait_in(b):
        pltpu.make_async_copy(row_hbm.at[pl.ds(0, CHUNK)], rows[b], sems_r[b]).wait()
        pltpu.make_async_copy(col_hbm.at[pl.ds(0, CHUNK)], cols[b], sems_c[b]).wait()
        pltpu.make_async_copy(dist_hbm.at[pl.ds(0, CHUNK)], dists[b], sems_d[b]).wait()

    def start_out(ci, b):
        base = tile_base + ci * CHUNK
        pltpu.async_copy(outs[b], out_hbm.at[pl.ds(base, CHUNK)], sems_out[b])

    def wait_out(b):
        pltpu.make_async_copy(outs[b], out_hbm.at[pl.ds(0, CHUNK)], sems_out[b]).wait()

    def compute(b):
        row_b, col_b, dist_b, out_b = rows[b], cols[b], dists[b], outs[b]

        @plsc.parallel_loop(0, CHUNK, 16, unroll=5)
        def _(i):
            s = pl.ds(i, 16)
            r = row_b[s]
            c = col_b[s]
            d = dist_b[s]
            z_i = plsc.load_gather(z_v, [r])
            z_j = plsc.load_gather(z_v, [c])
            p = z_i * NTP + z_j
            delta = plsc.load_gather(dtab, [p])
            aa = plsc.load_gather(atab, [p])
            a = aa * d + K0
            x = jnp.abs(a)
            t = 1.0 / (1.0 + _AP * x)
            poly = ((_A3 * t + _A2) * t + _A1) * t
            y = 1.0 - poly * jnp.exp(-(a * a))
            sbit = lax.bitcast_convert_type(a, jnp.int32) & _SIGN
            erf = lax.bitcast_convert_type(
                lax.bitcast_convert_type(y, jnp.int32) ^ sbit, jnp.float32)
            out_b[s] = delta * (1.0 + erf)

    start_in(0, 0)
    z_copy.wait()

    def pair_body(g, carry):
        c0 = 2 * g
        start_in(c0 + 1, 1)
        wait_in(0)

        @pl.when(g > 0)
        def _():
            wait_out(0)

        compute(0)
        start_out(c0, 0)
        start_in(c0 + 2, 0)
        wait_in(1)

        @pl.when(g > 0)
        def _():
            wait_out(1)

        compute(1)
        start_out(c0 + 1, 1)
        return carry

    lax.fori_loop(0, N_PAIRS, pair_body, 0)

    wait_in(0)
    wait_out(0)
    compute(0)
    start_out(N_CHUNKS - 1, 0)
    wait_out(0)
    wait_out(1)


_mesh = plsc.VectorSubcoreMesh(core_axis_name="c", subcore_axis_name="s")

_edge_kernel = functools.partial(
    pl.kernel,
    out_type=jax.ShapeDtypeStruct((N_EDGES,), jnp.float32),
    mesh=_mesh,
    compiler_params=pltpu.CompilerParams(needs_layout_passes=False),
    scratch_types=[
        pltpu.VMEM((N_NODES,), jnp.int32),
        pltpu.VMEM((NTP,), jnp.float32),
        pltpu.VMEM((NTP,), jnp.float32),
        pltpu.VMEM((NTP,), jnp.float32),
        pltpu.VMEM((NT * NTP,), jnp.float32),
        pltpu.VMEM((NT * NTP,), jnp.float32),
        pltpu.VMEM((CHUNK,), jnp.int32),
        pltpu.VMEM((CHUNK,), jnp.int32),
        pltpu.VMEM((CHUNK,), jnp.int32),
        pltpu.VMEM((CHUNK,), jnp.int32),
        pltpu.VMEM((CHUNK,), jnp.float32),
        pltpu.VMEM((CHUNK,), jnp.float32),
        pltpu.VMEM((CHUNK,), jnp.float32),
        pltpu.VMEM((CHUNK,), jnp.float32),
        pltpu.SemaphoreType.DMA,
        pltpu.SemaphoreType.DMA,
        pltpu.SemaphoreType.DMA,
        pltpu.SemaphoreType.DMA,
        pltpu.SemaphoreType.DMA,
        pltpu.SemaphoreType.DMA,
        pltpu.SemaphoreType.DMA,
        pltpu.SemaphoreType.DMA,
        pltpu.SemaphoreType.DMA,
    ],
)(_body)


def kernel(z, dist, edge_index, en_table, radius_table, corr_table):
    row = edge_index[0]
    col = edge_index[1]
    en = jnp.pad(en_table[:, 0], (0, NTP - NT), constant_values=1.0)
    rad = jnp.pad(radius_table[:, 0], (0, NTP - NT), constant_values=1.0)
    corr = jnp.pad(corr_table[:, 0], (0, NTP - NT), constant_values=0.0)
    out = _edge_kernel(z, row, col, dist, en, rad, corr)
    return out[:, None]
